# baseline (speedup 1.0000x reference)
import functools

import jax
import jax.numpy as jnp
from jax import lax
from jax.experimental import pallas as pl
from jax.experimental.pallas import tpu as pltpu
from jax.experimental.pallas import tpu_sc as plsc

_EPS = 1e-5


def _node_body(ft_ref, xt_ref, wc_ref, wn_ref, wp_ref, ab_ref, a_ref, b_ref):
    ft = ft_ref[...]
    xt = xt_ref[...]
    wc = wc_ref[...]
    wn = wn_ref[...]
    wp = wp_ref[...]
    xp = jnp.dot(xt, wp, preferred_element_type=jnp.float32,
                 precision=lax.Precision.HIGHEST)
    fc = jnp.dot(ft, wc, preferred_element_type=jnp.float32,
                 precision=lax.Precision.HIGHEST)
    fn = jnp.dot(ft, wn, preferred_element_type=jnp.float32,
                 precision=lax.Precision.HIGHEST)
    a_ref[...] = fc - xp + ab_ref[...]
    b_ref[...] = fn + xp


def _batcher_pairs(n):
    pairs = []
    p = 1
    while p < n:
        kk = p
        while kk >= 1:
            for j in range(kk % p, n - kk, 2 * kk):
                for i in range(0, min(kk, n - j - kk)):
                    if (i + j) // (2 * p) == (i + j + kk) // (2 * p):
                        pairs.append((i + j, i + j + kk))
            kk //= 2
        p *= 2
    return pairs


def _knn_body(xq_ref, xall_ref, idx_ref, *, n, k):
    xq = xq_ref[...]
    xall = xall_ref[...]
    q = xq.shape[0]
    qn = jnp.sum(xq * xq, axis=1, keepdims=True)
    cn = jnp.sum(xall * xall, axis=0, keepdims=True)
    qncn = qn + cn
    cross = xq[:, 0:1] * xall[0:1, :]
    cross += xq[:, 1:2] * xall[1:2, :]
    cross += xq[:, 2:3] * xall[2:3, :]
    dist = qncn - 2.0 * cross

    g_n = k
    w = n // g_n
    cols = lax.broadcasted_iota(jnp.int32, (q, w), 1)
    vals = [dist[:, g * w:(g + 1) * w] for g in range(g_n)]
    idxs = [cols + g * w for g in range(g_n)]
    for a, b in _batcher_pairs(g_n):
        lt = vals[a] <= vals[b]
        va = jnp.where(lt, vals[a], vals[b])
        vb = jnp.where(lt, vals[b], vals[a])
        ia = jnp.where(lt, idxs[a], idxs[b])
        ib = jnp.where(lt, idxs[b], idxs[a])
        vals[a], vals[b] = va, vb
        idxs[a], idxs[b] = ia, ib

    dcap = 4
    cnt = jnp.zeros((q, w), jnp.int32)
    for t in range(k):
        m = jnp.min(vals[0], axis=1, keepdims=True)
        c = jnp.min(jnp.where(vals[0] == m, cols, w), axis=1, keepdims=True)
        mask = cols == c
        gi = jnp.min(jnp.where(mask, idxs[0], n), axis=1, keepdims=True)
        idx_ref[:, t:t + 1] = gi
        cnt = cnt + mask.astype(jnp.int32)
        depth = min(dcap, k - 1 - t)
        for j in range(depth):
            vals[j] = jnp.where(mask, vals[j + 1], vals[j])
            idxs[j] = jnp.where(mask, idxs[j + 1], idxs[j])
        vals[depth] = jnp.where(mask, jnp.inf, vals[depth])

    deficient = jnp.any(cnt >= dcap + 1)

    @pl.when(deficient)
    def _exact_fallback():
        cols_full = lax.broadcasted_iota(jnp.int32, (q, n), 1)
        work = dist
        for t in range(k):
            m = jnp.min(work, axis=1, keepdims=True)
            j = jnp.min(jnp.where(work == m, cols_full, n), axis=1,
                        keepdims=True)
            idx_ref[:, t:t + 1] = j
            work = jnp.where(cols_full == j, jnp.inf, work)


def _edge_body(g_ref, a_ref, w2_ref, b2_ref, wa1_ref, ba1_ref, wa2_ref,
               out_ref, *, q, k):
    a = a_ref[...]
    w2 = w2_ref[...]
    b2 = b2_ref[...]
    wa1 = wa1_ref[...]
    ba1 = ba1_ref[...]
    wa2 = wa2_ref[...]
    h2s = []
    ls = []
    for t in range(k):
        g = g_ref[pl.ds(t * q, q), :]
        h1 = jnp.maximum(a + g, 0.0)
        h2 = jnp.maximum(
            jnp.dot(h1, w2, preferred_element_type=jnp.float32,
                    precision=lax.Precision.HIGHEST) + b2, 0.0)
        a1 = jnp.maximum(
            jnp.dot(h2, wa1, preferred_element_type=jnp.float32,
                    precision=lax.Precision.HIGHEST) + ba1, 0.0)
        l = jnp.dot(a1, wa2, preferred_element_type=jnp.float32,
                    precision=lax.Precision.HIGHEST)
        h2s.append(h2)
        ls.append(l)
    logits = jnp.concatenate(ls, axis=1)
    mx = jnp.max(logits, axis=1, keepdims=True)
    e = jnp.exp(logits - mx)
    attn = e / jnp.sum(e, axis=1, keepdims=True)
    out = h2s[0] * attn[:, 0:1]
    for t in range(1, k):
        out = out + h2s[t] * attn[:, t:t + 1]
    out_ref[...] = out


def _sc_gather(table, idx_flat, e_pad, d, window):
    mesh = plsc.VectorSubcoreMesh(core_axis_name="c", subcore_axis_name="s")

    @functools.partial(
        pl.kernel,
        out_type=jax.ShapeDtypeStruct((e_pad, d), jnp.float32),
        mesh=mesh)
    def k(x_hbm, i_hbm, o_hbm):
        def body(i_vmem, o_vmem):
            pltpu.sync_copy(x_hbm.at[i_vmem.at[0]], o_vmem)

        pltpu.emit_pipeline(
            body,
            grid=(e_pad // window,),
            in_specs=[pl.BlockSpec((1, window), lambda i: (0, i))],
            out_specs=[pl.BlockSpec((window, d), lambda i: (i, 0))],
            core_axis_name=("c", "s"),
            dimension_semantics=(pltpu.PARALLEL,),
        )(i_hbm, o_hbm)

    return k(table, idx_flat)


def kernel(xyz, features, W1, b1, g1, be1, rm1, rv1, W2, b2, g2, be2, rm2,
           rv2, Wa1, ba1, ga1, bea1, rma1, rva1, Wa2, ba2):
    n = xyz.shape[2]
    c = features.shape[1]
    out_c = W1.shape[0]
    k = 16
    mid = Wa1.shape[0]

    xyz_cn = xyz[0]
    xyz_nt = xyz_cn.T
    feat_nt = features[0].T

    f32 = jnp.float32

    s1 = g1 * lax.rsqrt(rv1 + _EPS)
    t1 = be1 - rm1 * s1
    s2 = g2 * lax.rsqrt(rv2 + _EPS)
    t2 = be2 - rm2 * s2
    sa = ga1 * lax.rsqrt(rva1 + _EPS)
    ta = bea1 - rma1 * sa
    wc = W1[:, :c].T * s1[None, :]
    wn = W1[:, c:2 * c].T * s1[None, :]
    wp = W1[:, 2 * c:].T * s1[None, :]
    ab = (b1 * s1 + t1)[None, :]
    w2 = W2.T * s2[None, :]
    b2f = (b2 * s2 + t2)[None, :]
    wa1 = Wa1.T * sa[None, :]
    ba1f = (ba1 * sa + ta)[None, :]
    wa2 = Wa2.T

    qb = 400
    nb = n // qb
    a_p, b_p = pl.pallas_call(
        _node_body,
        grid=(nb,),
        in_specs=[
            pl.BlockSpec((qb, c), lambda i: (i, 0)),
            pl.BlockSpec((qb, 3), lambda i: (i, 0)),
            pl.BlockSpec((c, out_c), lambda i: (0, 0)),
            pl.BlockSpec((c, out_c), lambda i: (0, 0)),
            pl.BlockSpec((3, out_c), lambda i: (0, 0)),
            pl.BlockSpec((1, out_c), lambda i: (0, 0)),
        ],
        out_specs=[
            pl.BlockSpec((qb, out_c), lambda i: (i, 0)),
            pl.BlockSpec((qb, out_c), lambda i: (i, 0)),
        ],
        out_shape=[
            jax.ShapeDtypeStruct((n, out_c), f32),
            jax.ShapeDtypeStruct((n, out_c), f32),
        ],
    )(feat_nt, xyz_nt, wc, wn, wp, ab)

    half = n // 2
    qk = 200
    nbk = half // qk
    e_h = half * k
    window = 128
    gran = 32 * window
    e_pad = ((e_h + gran - 1) // gran) * gran
    outs = []
    for h in range(2):
        idx = pl.pallas_call(
            functools.partial(_knn_body, n=n, k=k),
            grid=(nbk,),
            in_specs=[
                pl.BlockSpec((qk, 3), lambda i, o=h * nbk: (i + o, 0)),
                pl.BlockSpec((3, n), lambda i: (0, 0)),
            ],
            out_specs=pl.BlockSpec((qk, k), lambda i: (i, 0)),
            out_shape=jax.ShapeDtypeStruct((half, k), jnp.int32),
        )(xyz_nt, xyz_cn)

        idx_bm = idx.reshape(nbk, qk, k).transpose(0, 2, 1).reshape(1, e_h)
        idx_flat = jnp.concatenate(
            [idx_bm, jnp.zeros((1, e_pad - e_h), jnp.int32)], axis=1)
        g_rows = _sc_gather(b_p, idx_flat, e_pad, out_c, window)

        out_h = pl.pallas_call(
            functools.partial(_edge_body, q=qk, k=k),
            grid=(nbk,),
            in_specs=[
                pl.BlockSpec((k * qk, out_c), lambda i: (i, 0)),
                pl.BlockSpec((qk, out_c), lambda i, o=h * nbk: (i + o, 0)),
                pl.BlockSpec((out_c, out_c), lambda i: (0, 0)),
                pl.BlockSpec((1, out_c), lambda i: (0, 0)),
                pl.BlockSpec((out_c, mid), lambda i: (0, 0)),
                pl.BlockSpec((1, mid), lambda i: (0, 0)),
                pl.BlockSpec((mid, 1), lambda i: (0, 0)),
            ],
            out_specs=pl.BlockSpec((qk, out_c), lambda i: (i, 0)),
            out_shape=jax.ShapeDtypeStruct((half, out_c), f32),
        )(g_rows, a_p, w2, b2f, wa1, ba1f, wa2)
        outs.append(out_h)

    out_t = jnp.concatenate(outs, axis=0)
    return out_t.T[None]


# speedup vs baseline: 1.0005x; 1.0005x over previous
import functools

import jax
import jax.numpy as jnp
from jax import lax
from jax.experimental import pallas as pl
from jax.experimental.pallas import tpu as pltpu
from jax.experimental.pallas import tpu_sc as plsc

_EPS = 1e-5


def _node_body(ft_ref, xt_ref, wc_ref, wn_ref, wp_ref, ab_ref, a_ref, b_ref):
    ft = ft_ref[...]
    xt = xt_ref[...]
    wc = wc_ref[...]
    wn = wn_ref[...]
    wp = wp_ref[...]
    xp = jnp.dot(xt, wp, preferred_element_type=jnp.float32,
                 precision=lax.Precision.HIGHEST)
    fc = jnp.dot(ft, wc, preferred_element_type=jnp.float32,
                 precision=lax.Precision.HIGHEST)
    fn = jnp.dot(ft, wn, preferred_element_type=jnp.float32,
                 precision=lax.Precision.HIGHEST)
    a_ref[...] = fc - xp + ab_ref[...]
    b_ref[...] = fn + xp


def _batcher_pairs(n):
    pairs = []
    p = 1
    while p < n:
        kk = p
        while kk >= 1:
            for j in range(kk % p, n - kk, 2 * kk):
                for i in range(0, min(kk, n - j - kk)):
                    if (i + j) // (2 * p) == (i + j + kk) // (2 * p):
                        pairs.append((i + j, i + j + kk))
            kk //= 2
        p *= 2
    return pairs


def _knn_body(xq_ref, xall_ref, idx_ref, *, n, k):
    xq = xq_ref[...]
    xall = xall_ref[...]
    q = xq.shape[0]
    qn = jnp.sum(xq * xq, axis=1, keepdims=True)
    cn = jnp.sum(xall * xall, axis=0, keepdims=True)
    cross = xq[:, 0:1] * xall[0:1, :]
    cross += xq[:, 1:2] * xall[1:2, :]
    cross += xq[:, 2:3] * xall[2:3, :]
    dist = (qn + cn) - 2.0 * cross

    g_n = k
    w = n // g_n
    cols = lax.broadcasted_iota(jnp.int32, (q, w), 1)
    vals = [dist[:, g * w:(g + 1) * w] for g in range(g_n)]
    idxs = [cols + g * w for g in range(g_n)]
    for a, b in _batcher_pairs(g_n):
        lt = vals[a] <= vals[b]
        va = jnp.where(lt, vals[a], vals[b])
        vb = jnp.where(lt, vals[b], vals[a])
        ia = jnp.where(lt, idxs[a], idxs[b])
        ib = jnp.where(lt, idxs[b], idxs[a])
        vals[a], vals[b] = va, vb
        idxs[a], idxs[b] = ia, ib

    dcap = 4
    cnt = jnp.zeros((q, w), jnp.int32)
    for t in range(k):
        m = jnp.min(vals[0], axis=1, keepdims=True)
        c = jnp.min(jnp.where(vals[0] == m, cols, w), axis=1, keepdims=True)
        mask = cols == c
        gi = jnp.min(jnp.where(mask, idxs[0], n), axis=1, keepdims=True)
        idx_ref[:, t:t + 1] = gi
        cnt = cnt + mask.astype(jnp.int32)
        depth = min(dcap, k - 1 - t)
        for j in range(depth):
            vals[j] = jnp.where(mask, vals[j + 1], vals[j])
            idxs[j] = jnp.where(mask, idxs[j + 1], idxs[j])
        vals[depth] = jnp.where(mask, jnp.inf, vals[depth])

    deficient = jnp.any(cnt >= dcap + 1)

    @pl.when(deficient)
    def _exact_fallback():
        cols_full = lax.broadcasted_iota(jnp.int32, (q, n), 1)
        cr = xq[:, 0:1] * xall[0:1, :]
        cr += xq[:, 1:2] * xall[1:2, :]
        cr += xq[:, 2:3] * xall[2:3, :]
        work = (qn + cn) - 2.0 * cr
        for t in range(k):
            m = jnp.min(work, axis=1, keepdims=True)
            j = jnp.min(jnp.where(work == m, cols_full, n), axis=1,
                        keepdims=True)
            idx_ref[:, t:t + 1] = j
            work = jnp.where(cols_full == j, jnp.inf, work)


def _edge_body(g_ref, a_ref, w2_ref, b2_ref, wa1_ref, ba1_ref, wa2_ref,
               out_ref, *, q, k):
    a = a_ref[...]
    w2 = w2_ref[...]
    b2 = b2_ref[...]
    wa1 = wa1_ref[...]
    ba1 = ba1_ref[...]
    wa2 = wa2_ref[...]
    h2s = []
    ls = []
    for t in range(k):
        g = g_ref[pl.ds(t * q, q), :]
        h1 = jnp.maximum(a + g, 0.0)
        h2 = jnp.maximum(
            jnp.dot(h1, w2, preferred_element_type=jnp.float32,
                    precision=lax.Precision.HIGHEST) + b2, 0.0)
        a1 = jnp.maximum(
            jnp.dot(h2, wa1, preferred_element_type=jnp.float32,
                    precision=lax.Precision.HIGHEST) + ba1, 0.0)
        l = jnp.dot(a1, wa2, preferred_element_type=jnp.float32,
                    precision=lax.Precision.HIGHEST)
        h2s.append(h2)
        ls.append(l)
    logits = jnp.concatenate(ls, axis=1)
    mx = jnp.max(logits, axis=1, keepdims=True)
    e = jnp.exp(logits - mx)
    attn = e / jnp.sum(e, axis=1, keepdims=True)
    out = h2s[0] * attn[:, 0:1]
    for t in range(1, k):
        out = out + h2s[t] * attn[:, t:t + 1]
    out_ref[...] = out


def _sc_gather(table, idx_flat, e_pad, d, window):
    mesh = plsc.VectorSubcoreMesh(core_axis_name="c", subcore_axis_name="s")

    @functools.partial(
        pl.kernel,
        out_type=jax.ShapeDtypeStruct((e_pad, d), jnp.float32),
        mesh=mesh)
    def k(x_hbm, i_hbm, o_hbm):
        def body(i_vmem, o_vmem):
            pltpu.sync_copy(x_hbm.at[i_vmem.at[0]], o_vmem)

        pltpu.emit_pipeline(
            body,
            grid=(e_pad // window,),
            in_specs=[pl.BlockSpec((1, window), lambda i: (0, i))],
            out_specs=[pl.BlockSpec((window, d), lambda i: (i, 0))],
            core_axis_name=("c", "s"),
            dimension_semantics=(pltpu.PARALLEL,),
        )(i_hbm, o_hbm)

    return k(table, idx_flat)


def kernel(xyz, features, W1, b1, g1, be1, rm1, rv1, W2, b2, g2, be2, rm2,
           rv2, Wa1, ba1, ga1, bea1, rma1, rva1, Wa2, ba2):
    n = xyz.shape[2]
    c = features.shape[1]
    out_c = W1.shape[0]
    k = 16
    mid = Wa1.shape[0]

    xyz_cn = xyz[0]
    xyz_nt = xyz_cn.T
    feat_nt = features[0].T

    f32 = jnp.float32

    s1 = g1 * lax.rsqrt(rv1 + _EPS)
    t1 = be1 - rm1 * s1
    s2 = g2 * lax.rsqrt(rv2 + _EPS)
    t2 = be2 - rm2 * s2
    sa = ga1 * lax.rsqrt(rva1 + _EPS)
    ta = bea1 - rma1 * sa
    wc = W1[:, :c].T * s1[None, :]
    wn = W1[:, c:2 * c].T * s1[None, :]
    wp = W1[:, 2 * c:].T * s1[None, :]
    ab = (b1 * s1 + t1)[None, :]
    w2 = W2.T * s2[None, :]
    b2f = (b2 * s2 + t2)[None, :]
    wa1 = Wa1.T * sa[None, :]
    ba1f = (ba1 * sa + ta)[None, :]
    wa2 = Wa2.T

    qb = 400
    nb = n // qb
    a_p, b_p = pl.pallas_call(
        _node_body,
        grid=(nb,),
        in_specs=[
            pl.BlockSpec((qb, c), lambda i: (i, 0)),
            pl.BlockSpec((qb, 3), lambda i: (i, 0)),
            pl.BlockSpec((c, out_c), lambda i: (0, 0)),
            pl.BlockSpec((c, out_c), lambda i: (0, 0)),
            pl.BlockSpec((3, out_c), lambda i: (0, 0)),
            pl.BlockSpec((1, out_c), lambda i: (0, 0)),
        ],
        out_specs=[
            pl.BlockSpec((qb, out_c), lambda i: (i, 0)),
            pl.BlockSpec((qb, out_c), lambda i: (i, 0)),
        ],
        out_shape=[
            jax.ShapeDtypeStruct((n, out_c), f32),
            jax.ShapeDtypeStruct((n, out_c), f32),
        ],
    )(feat_nt, xyz_nt, wc, wn, wp, ab)

    half = n // 2
    qk = 200
    nbk = half // qk
    e_h = half * k
    window = 128
    gran = 32 * window
    e_pad = ((e_h + gran - 1) // gran) * gran
    outs = []
    for h in range(2):
        idx = pl.pallas_call(
            functools.partial(_knn_body, n=n, k=k),
            grid=(nbk,),
            in_specs=[
                pl.BlockSpec((qk, 3), lambda i, o=h * nbk: (i + o, 0)),
                pl.BlockSpec((3, n), lambda i: (0, 0)),
            ],
            out_specs=pl.BlockSpec((qk, k), lambda i: (i, 0)),
            out_shape=jax.ShapeDtypeStruct((half, k), jnp.int32),
        )(xyz_nt, xyz_cn)

        idx_bm = idx.reshape(nbk, qk, k).transpose(0, 2, 1).reshape(1, e_h)
        idx_flat = jnp.concatenate(
            [idx_bm, jnp.zeros((1, e_pad - e_h), jnp.int32)], axis=1)
        g_rows = _sc_gather(b_p, idx_flat, e_pad, out_c, window)

        out_h = pl.pallas_call(
            functools.partial(_edge_body, q=qk, k=k),
            grid=(nbk,),
            in_specs=[
                pl.BlockSpec((k * qk, out_c), lambda i: (i, 0)),
                pl.BlockSpec((qk, out_c), lambda i, o=h * nbk: (i + o, 0)),
                pl.BlockSpec((out_c, out_c), lambda i: (0, 0)),
                pl.BlockSpec((1, out_c), lambda i: (0, 0)),
                pl.BlockSpec((out_c, mid), lambda i: (0, 0)),
                pl.BlockSpec((1, mid), lambda i: (0, 0)),
                pl.BlockSpec((mid, 1), lambda i: (0, 0)),
            ],
            out_specs=pl.BlockSpec((qk, out_c), lambda i: (i, 0)),
            out_shape=jax.ShapeDtypeStruct((half, out_c), f32),
        )(g_rows, a_p, w2, b2f, wa1, ba1f, wa2)
        outs.append(out_h)

    out_t = jnp.concatenate(outs, axis=0)
    return out_t.T[None]


# speedup vs baseline: 1.4900x; 1.4892x over previous
import functools

import jax
import jax.numpy as jnp
from jax import lax
from jax.experimental import pallas as pl
from jax.experimental.pallas import tpu as pltpu
from jax.experimental.pallas import tpu_sc as plsc

_EPS = 1e-5


def _node_body(ft_ref, xt_ref, wc_ref, wn_ref, wp_ref, ab_ref, a_ref, b_ref):
    ft = ft_ref[...]
    xt = xt_ref[...]
    wc = wc_ref[...]
    wn = wn_ref[...]
    wp = wp_ref[...]
    xp = jnp.dot(xt, wp, preferred_element_type=jnp.float32,
                 precision=lax.Precision.HIGHEST)
    fc = jnp.dot(ft, wc, preferred_element_type=jnp.float32,
                 precision=lax.Precision.HIGHEST)
    fn = jnp.dot(ft, wn, preferred_element_type=jnp.float32,
                 precision=lax.Precision.HIGHEST)
    a_ref[...] = fc - xp + ab_ref[...]
    b_ref[...] = fn + xp


def _batcher_pairs(n):
    pairs = []
    p = 1
    while p < n:
        kk = p
        while kk >= 1:
            for j in range(kk % p, n - kk, 2 * kk):
                for i in range(0, min(kk, n - j - kk)):
                    if (i + j) // (2 * p) == (i + j + kk) // (2 * p):
                        pairs.append((i + j, i + j + kk))
            kk //= 2
        p *= 2
    return pairs


def _selection_pairs(n, m):
    net = _batcher_pairs(n)
    needed = set(range(m))
    keep = []
    for ce in reversed(net):
        a, b = ce
        if a in needed or b in needed:
            keep.append(ce)
            needed.add(a)
            needed.add(b)
    return list(reversed(keep))


def _knn_body(xq_ref, xall_ref, idx_ref, *, n, k):
    xq = xq_ref[...]
    xall = xall_ref[...]
    q = xq.shape[0]
    qn = jnp.sum(xq * xq, axis=1, keepdims=True)
    cn = jnp.sum(xall * xall, axis=0, keepdims=True)
    cross = xq[:, 0:1] * xall[0:1, :]
    cross += xq[:, 1:2] * xall[1:2, :]
    cross += xq[:, 2:3] * xall[2:3, :]
    dist = (qn + cn) - 2.0 * cross

    g_n = k
    dcap = 4
    w = n // g_n
    cols = lax.broadcasted_iota(jnp.int32, (q, w), 1)
    vals = [dist[:, g * w:(g + 1) * w] for g in range(g_n)]
    idxs = [cols + g * w for g in range(g_n)]
    for a, b in _selection_pairs(g_n, dcap + 1):
        lt = vals[a] <= vals[b]
        va = jnp.where(lt, vals[a], vals[b])
        vb = jnp.where(lt, vals[b], vals[a])
        ia = jnp.where(lt, idxs[a], idxs[b])
        ib = jnp.where(lt, idxs[b], idxs[a])
        vals[a], vals[b] = va, vb
        idxs[a], idxs[b] = ia, ib

    cnt = jnp.zeros((q, w), jnp.int32)
    for t in range(k):
        m = jnp.min(vals[0], axis=1, keepdims=True)
        c = jnp.min(jnp.where(vals[0] == m, cols, w), axis=1, keepdims=True)
        mask = cols == c
        gi = jnp.min(jnp.where(mask, idxs[0], n), axis=1, keepdims=True)
        idx_ref[:, t:t + 1] = gi
        cnt = cnt + mask.astype(jnp.int32)
        depth = min(dcap, k - 1 - t)
        for j in range(depth):
            vals[j] = jnp.where(mask, vals[j + 1], vals[j])
            idxs[j] = jnp.where(mask, idxs[j + 1], idxs[j])
        vals[depth] = jnp.where(mask, jnp.inf, vals[depth])

    deficient = jnp.any(cnt >= dcap + 1)

    @pl.when(deficient)
    def _exact_fallback():
        cr = xq[:, 0:1] * xall[0:1, :]
        cr += xq[:, 1:2] * xall[1:2, :]
        cr += xq[:, 2:3] * xall[2:3, :]
        work0 = (qn + cn) - 2.0 * cr
        js0 = jnp.zeros((q, k), jnp.int32)
        kio = lax.broadcasted_iota(jnp.int32, (q, k), 1)

        def body(t, carry):
            work, js = carry
            cf = lax.broadcasted_iota(jnp.int32, (q, n), 1)
            m = jnp.min(work, axis=1, keepdims=True)
            j = jnp.min(jnp.where(work == m, cf, n), axis=1, keepdims=True)
            js = jnp.where(kio == t, j, js)
            work = jnp.where(cf == j, jnp.inf, work)
            return work, js

        _, js = lax.fori_loop(0, k, body, (work0, js0))
        idx_ref[...] = js


def _edge_body(g_ref, a_ref, w2_ref, b2_ref, wa1_ref, ba1_ref, wa2_ref,
               out_ref, *, q, k):
    a = a_ref[...]
    w2 = w2_ref[...]
    b2 = b2_ref[...]
    wa1 = wa1_ref[...]
    ba1 = ba1_ref[...]
    wa2 = wa2_ref[...]
    h2s = []
    ls = []
    for t in range(k):
        g = g_ref[pl.ds(t * q, q), :]
        h1 = jnp.maximum(a + g, 0.0)
        h2 = jnp.maximum(
            jnp.dot(h1, w2, preferred_element_type=jnp.float32,
                    precision=lax.Precision.HIGHEST) + b2, 0.0)
        a1 = jnp.maximum(
            jnp.dot(h2, wa1, preferred_element_type=jnp.float32,
                    precision=lax.Precision.HIGHEST) + ba1, 0.0)
        l = jnp.dot(a1, wa2, preferred_element_type=jnp.float32,
                    precision=lax.Precision.HIGHEST)
        h2s.append(h2)
        ls.append(l)
    logits = jnp.concatenate(ls, axis=1)
    mx = jnp.max(logits, axis=1, keepdims=True)
    e = jnp.exp(logits - mx)
    attn = e / jnp.sum(e, axis=1, keepdims=True)
    out = h2s[0] * attn[:, 0:1]
    for t in range(1, k):
        out = out + h2s[t] * attn[:, t:t + 1]
    out_ref[...] = out


def _sc_gather(table, idx_flat, e_pad, d, window):
    mesh = plsc.VectorSubcoreMesh(core_axis_name="c", subcore_axis_name="s")

    @functools.partial(
        pl.kernel,
        out_type=jax.ShapeDtypeStruct((e_pad, d), jnp.float32),
        mesh=mesh)
    def k(x_hbm, i_hbm, o_hbm):
        def body(i_vmem, o_vmem):
            pltpu.sync_copy(x_hbm.at[i_vmem.at[0]], o_vmem)

        pltpu.emit_pipeline(
            body,
            grid=(e_pad // window,),
            in_specs=[pl.BlockSpec((1, window), lambda i: (0, i))],
            out_specs=[pl.BlockSpec((window, d), lambda i: (i, 0))],
            core_axis_name=("c", "s"),
            dimension_semantics=(pltpu.PARALLEL,),
        )(i_hbm, o_hbm)

    return k(table, idx_flat)


def kernel(xyz, features, W1, b1, g1, be1, rm1, rv1, W2, b2, g2, be2, rm2,
           rv2, Wa1, ba1, ga1, bea1, rma1, rva1, Wa2, ba2):
    n = xyz.shape[2]
    c = features.shape[1]
    out_c = W1.shape[0]
    k = 16
    mid = Wa1.shape[0]

    xyz_cn = xyz[0]
    xyz_nt = xyz_cn.T
    feat_nt = features[0].T

    f32 = jnp.float32

    s1 = g1 * lax.rsqrt(rv1 + _EPS)
    t1 = be1 - rm1 * s1
    s2 = g2 * lax.rsqrt(rv2 + _EPS)
    t2 = be2 - rm2 * s2
    sa = ga1 * lax.rsqrt(rva1 + _EPS)
    ta = bea1 - rma1 * sa
    wc = W1[:, :c].T * s1[None, :]
    wn = W1[:, c:2 * c].T * s1[None, :]
    wp = W1[:, 2 * c:].T * s1[None, :]
    ab = (b1 * s1 + t1)[None, :]
    w2 = W2.T * s2[None, :]
    b2f = (b2 * s2 + t2)[None, :]
    wa1 = Wa1.T * sa[None, :]
    ba1f = (ba1 * sa + ta)[None, :]
    wa2 = Wa2.T

    qb = 400
    nb = n // qb
    a_p, b_p = pl.pallas_call(
        _node_body,
        grid=(nb,),
        in_specs=[
            pl.BlockSpec((qb, c), lambda i: (i, 0)),
            pl.BlockSpec((qb, 3), lambda i: (i, 0)),
            pl.BlockSpec((c, out_c), lambda i: (0, 0)),
            pl.BlockSpec((c, out_c), lambda i: (0, 0)),
            pl.BlockSpec((3, out_c), lambda i: (0, 0)),
            pl.BlockSpec((1, out_c), lambda i: (0, 0)),
        ],
        out_specs=[
            pl.BlockSpec((qb, out_c), lambda i: (i, 0)),
            pl.BlockSpec((qb, out_c), lambda i: (i, 0)),
        ],
        out_shape=[
            jax.ShapeDtypeStruct((n, out_c), f32),
            jax.ShapeDtypeStruct((n, out_c), f32),
        ],
    )(feat_nt, xyz_nt, wc, wn, wp, ab)

    half = n // 2
    qk = 200
    nbk = half // qk
    e_h = half * k
    window = 128
    gran = 32 * window
    e_pad = ((e_h + gran - 1) // gran) * gran
    outs = []
    for h in range(2):
        idx = pl.pallas_call(
            functools.partial(_knn_body, n=n, k=k),
            grid=(nbk,),
            in_specs=[
                pl.BlockSpec((qk, 3), lambda i, o=h * nbk: (i + o, 0)),
                pl.BlockSpec((3, n), lambda i: (0, 0)),
            ],
            out_specs=pl.BlockSpec((qk, k), lambda i: (i, 0)),
            out_shape=jax.ShapeDtypeStruct((half, k), jnp.int32),
        )(xyz_nt, xyz_cn)

        idx_bm = idx.reshape(nbk, qk, k).transpose(0, 2, 1).reshape(1, e_h)
        idx_flat = jnp.concatenate(
            [idx_bm, jnp.zeros((1, e_pad - e_h), jnp.int32)], axis=1)
        g_rows = _sc_gather(b_p, idx_flat, e_pad, out_c, window)

        out_h = pl.pallas_call(
            functools.partial(_edge_body, q=qk, k=k),
            grid=(nbk,),
            in_specs=[
                pl.BlockSpec((k * qk, out_c), lambda i: (i, 0)),
                pl.BlockSpec((qk, out_c), lambda i, o=h * nbk: (i + o, 0)),
                pl.BlockSpec((out_c, out_c), lambda i: (0, 0)),
                pl.BlockSpec((1, out_c), lambda i: (0, 0)),
                pl.BlockSpec((out_c, mid), lambda i: (0, 0)),
                pl.BlockSpec((1, mid), lambda i: (0, 0)),
                pl.BlockSpec((mid, 1), lambda i: (0, 0)),
            ],
            out_specs=pl.BlockSpec((qk, out_c), lambda i: (i, 0)),
            out_shape=jax.ShapeDtypeStruct((half, out_c), f32),
        )(g_rows, a_p, w2, b2f, wa1, ba1f, wa2)
        outs.append(out_h)

    out_t = jnp.concatenate(outs, axis=0)
    return out_t.T[None]


# speedup vs baseline: 1.4902x; 1.0001x over previous
import functools

import jax
import jax.numpy as jnp
from jax import lax
from jax.experimental import pallas as pl
from jax.experimental.pallas import tpu as pltpu
from jax.experimental.pallas import tpu_sc as plsc

_EPS = 1e-5


def _node_body(ft_ref, xt_ref, wc_ref, wn_ref, wp_ref, ab_ref, a_ref, b_ref):
    ft = ft_ref[...]
    xt = xt_ref[...]
    wc = wc_ref[...]
    wn = wn_ref[...]
    wp = wp_ref[...]
    xp = jnp.dot(xt, wp, preferred_element_type=jnp.float32,
                 precision=lax.Precision.HIGHEST)
    fc = jnp.dot(ft, wc, preferred_element_type=jnp.float32,
                 precision=lax.Precision.HIGHEST)
    fn = jnp.dot(ft, wn, preferred_element_type=jnp.float32,
                 precision=lax.Precision.HIGHEST)
    a_ref[...] = fc - xp + ab_ref[...]
    b_ref[...] = fn + xp


def _batcher_pairs(n):
    pairs = []
    p = 1
    while p < n:
        kk = p
        while kk >= 1:
            for j in range(kk % p, n - kk, 2 * kk):
                for i in range(0, min(kk, n - j - kk)):
                    if (i + j) // (2 * p) == (i + j + kk) // (2 * p):
                        pairs.append((i + j, i + j + kk))
            kk //= 2
        p *= 2
    return pairs


def _selection_pairs(n, m):
    net = _batcher_pairs(n)
    needed = set(range(m))
    keep = []
    for ce in reversed(net):
        a, b = ce
        if a in needed or b in needed:
            keep.append(ce)
            needed.add(a)
            needed.add(b)
    return list(reversed(keep))


def _knn_body(xq_ref, xall_ref, idx_ref, *, n, k):
    xq = xq_ref[...]
    xall = xall_ref[...]
    q = xq.shape[0]
    qn = jnp.sum(xq * xq, axis=1, keepdims=True)
    cn = jnp.sum(xall * xall, axis=0, keepdims=True)
    cross = xq[:, 0:1] * xall[0:1, :]
    cross += xq[:, 1:2] * xall[1:2, :]
    cross += xq[:, 2:3] * xall[2:3, :]
    dist = (qn + cn) - 2.0 * cross

    g_n = k
    dcap = 4
    w = n // g_n
    cols = lax.broadcasted_iota(jnp.int32, (q, w), 1)
    vals = [dist[:, g * w:(g + 1) * w] for g in range(g_n)]
    idxs = [cols + g * w for g in range(g_n)]
    for a, b in _selection_pairs(g_n, dcap + 1):
        lt = vals[a] <= vals[b]
        va = jnp.where(lt, vals[a], vals[b])
        vb = jnp.where(lt, vals[b], vals[a])
        ia = jnp.where(lt, idxs[a], idxs[b])
        ib = jnp.where(lt, idxs[b], idxs[a])
        vals[a], vals[b] = va, vb
        idxs[a], idxs[b] = ia, ib

    cnt = jnp.zeros((q, w), jnp.int32)
    for t in range(k):
        m = jnp.min(vals[0], axis=1, keepdims=True)
        c = jnp.min(jnp.where(vals[0] == m, cols, w), axis=1, keepdims=True)
        mask = cols == c
        gi = jnp.min(jnp.where(mask, idxs[0], n), axis=1, keepdims=True)
        idx_ref[:, t:t + 1] = gi
        cnt = cnt + mask.astype(jnp.int32)
        depth = min(dcap, k - 1 - t)
        for j in range(depth):
            vals[j] = jnp.where(mask, vals[j + 1], vals[j])
            idxs[j] = jnp.where(mask, idxs[j + 1], idxs[j])
        vals[depth] = jnp.where(mask, jnp.inf, vals[depth])

    deficient = jnp.any(cnt >= dcap + 1)

    @pl.when(deficient)
    def _exact_fallback():
        cr = xq[:, 0:1] * xall[0:1, :]
        cr += xq[:, 1:2] * xall[1:2, :]
        cr += xq[:, 2:3] * xall[2:3, :]
        work0 = (qn + cn) - 2.0 * cr
        js0 = jnp.zeros((q, k), jnp.int32)
        kio = lax.broadcasted_iota(jnp.int32, (q, k), 1)

        def body(t, carry):
            work, js = carry
            cf = lax.broadcasted_iota(jnp.int32, (q, n), 1)
            m = jnp.min(work, axis=1, keepdims=True)
            j = jnp.min(jnp.where(work == m, cf, n), axis=1, keepdims=True)
            js = jnp.where(kio == t, j, js)
            work = jnp.where(cf == j, jnp.inf, work)
            return work, js

        _, js = lax.fori_loop(0, k, body, (work0, js0))
        idx_ref[...] = js


def _edge_body(g_ref, a_ref, w2_ref, b2_ref, wa1_ref, ba1_ref, wa2_ref,
               out_ref, *, q, k):
    a = a_ref[...]
    w2 = w2_ref[...]
    b2 = b2_ref[...]
    wa1 = wa1_ref[...]
    ba1 = ba1_ref[...]
    wa2 = wa2_ref[...]
    h2s = []
    ls = []
    for t in range(k):
        g = g_ref[pl.ds(t * q, q), :]
        h1 = jnp.maximum(a + g, 0.0)
        h2 = jnp.maximum(
            jnp.dot(h1, w2, preferred_element_type=jnp.float32,
                    precision=lax.Precision.HIGHEST) + b2, 0.0)
        a1 = jnp.maximum(
            jnp.dot(h2, wa1, preferred_element_type=jnp.float32,
                    precision=lax.Precision.HIGHEST) + ba1, 0.0)
        l = jnp.dot(a1, wa2, preferred_element_type=jnp.float32,
                    precision=lax.Precision.HIGHEST)
        h2s.append(h2)
        ls.append(l)
    logits = jnp.concatenate(ls, axis=1)
    mx = jnp.max(logits, axis=1, keepdims=True)
    e = jnp.exp(logits - mx)
    attn = e / jnp.sum(e, axis=1, keepdims=True)
    out = h2s[0] * attn[:, 0:1]
    for t in range(1, k):
        out = out + h2s[t] * attn[:, t:t + 1]
    out_ref[...] = out


def _sc_gather(table, idx_flat, e_pad, d, window):
    mesh = plsc.VectorSubcoreMesh(core_axis_name="c", subcore_axis_name="s")

    @functools.partial(
        pl.kernel,
        out_type=jax.ShapeDtypeStruct((e_pad, d), jnp.float32),
        mesh=mesh)
    def k(x_hbm, i_hbm, o_hbm):
        def body(i_vmem, o_vmem):
            def inner(sem0, sem1):
                h0 = pltpu.async_copy(
                    x_hbm.at[i_vmem.at[0]], o_vmem.at[pl.ds(0, window)], sem0)
                h1 = pltpu.async_copy(
                    x_hbm.at[i_vmem.at[1]],
                    o_vmem.at[pl.ds(window, window)], sem1)
                h0.wait()
                h1.wait()

            pl.run_scoped(inner, pltpu.SemaphoreType.DMA,
                          pltpu.SemaphoreType.DMA)

        pltpu.emit_pipeline(
            body,
            grid=(e_pad // (2 * window),),
            in_specs=[pl.BlockSpec((2, window), lambda i: (i, 0))],
            out_specs=[pl.BlockSpec((2 * window, d), lambda i: (i, 0))],
            core_axis_name=("c", "s"),
            dimension_semantics=(pltpu.PARALLEL,),
        )(i_hbm, o_hbm)

    return k(table, idx_flat.reshape(e_pad // window, window))


def kernel(xyz, features, W1, b1, g1, be1, rm1, rv1, W2, b2, g2, be2, rm2,
           rv2, Wa1, ba1, ga1, bea1, rma1, rva1, Wa2, ba2):
    n = xyz.shape[2]
    c = features.shape[1]
    out_c = W1.shape[0]
    k = 16
    mid = Wa1.shape[0]

    xyz_cn = xyz[0]
    xyz_nt = xyz_cn.T
    feat_nt = features[0].T

    f32 = jnp.float32

    s1 = g1 * lax.rsqrt(rv1 + _EPS)
    t1 = be1 - rm1 * s1
    s2 = g2 * lax.rsqrt(rv2 + _EPS)
    t2 = be2 - rm2 * s2
    sa = ga1 * lax.rsqrt(rva1 + _EPS)
    ta = bea1 - rma1 * sa
    wc = W1[:, :c].T * s1[None, :]
    wn = W1[:, c:2 * c].T * s1[None, :]
    wp = W1[:, 2 * c:].T * s1[None, :]
    ab = (b1 * s1 + t1)[None, :]
    w2 = W2.T * s2[None, :]
    b2f = (b2 * s2 + t2)[None, :]
    wa1 = Wa1.T * sa[None, :]
    ba1f = (ba1 * sa + ta)[None, :]
    wa2 = Wa2.T

    qb = 400
    nb = n // qb
    a_p, b_p = pl.pallas_call(
        _node_body,
        grid=(nb,),
        in_specs=[
            pl.BlockSpec((qb, c), lambda i: (i, 0)),
            pl.BlockSpec((qb, 3), lambda i: (i, 0)),
            pl.BlockSpec((c, out_c), lambda i: (0, 0)),
            pl.BlockSpec((c, out_c), lambda i: (0, 0)),
            pl.BlockSpec((3, out_c), lambda i: (0, 0)),
            pl.BlockSpec((1, out_c), lambda i: (0, 0)),
        ],
        out_specs=[
            pl.BlockSpec((qb, out_c), lambda i: (i, 0)),
            pl.BlockSpec((qb, out_c), lambda i: (i, 0)),
        ],
        out_shape=[
            jax.ShapeDtypeStruct((n, out_c), f32),
            jax.ShapeDtypeStruct((n, out_c), f32),
        ],
    )(feat_nt, xyz_nt, wc, wn, wp, ab)

    half = n // 2
    qk = 200
    nbk = half // qk
    e_h = half * k
    window = 128
    gran = 32 * window
    e_pad = ((e_h + gran - 1) // gran) * gran
    outs = []
    for h in range(2):
        idx = pl.pallas_call(
            functools.partial(_knn_body, n=n, k=k),
            grid=(nbk,),
            in_specs=[
                pl.BlockSpec((qk, 3), lambda i, o=h * nbk: (i + o, 0)),
                pl.BlockSpec((3, n), lambda i: (0, 0)),
            ],
            out_specs=pl.BlockSpec((qk, k), lambda i: (i, 0)),
            out_shape=jax.ShapeDtypeStruct((half, k), jnp.int32),
        )(xyz_nt, xyz_cn)

        idx_bm = idx.reshape(nbk, qk, k).transpose(0, 2, 1).reshape(1, e_h)
        idx_flat = jnp.concatenate(
            [idx_bm, jnp.zeros((1, e_pad - e_h), jnp.int32)], axis=1)
        g_rows = _sc_gather(b_p, idx_flat, e_pad, out_c, window)

        out_h = pl.pallas_call(
            functools.partial(_edge_body, q=qk, k=k),
            grid=(nbk,),
            in_specs=[
                pl.BlockSpec((k * qk, out_c), lambda i: (i, 0)),
                pl.BlockSpec((qk, out_c), lambda i, o=h * nbk: (i + o, 0)),
                pl.BlockSpec((out_c, out_c), lambda i: (0, 0)),
                pl.BlockSpec((1, out_c), lambda i: (0, 0)),
                pl.BlockSpec((out_c, mid), lambda i: (0, 0)),
                pl.BlockSpec((1, mid), lambda i: (0, 0)),
                pl.BlockSpec((mid, 1), lambda i: (0, 0)),
            ],
            out_specs=pl.BlockSpec((qk, out_c), lambda i: (i, 0)),
            out_shape=jax.ShapeDtypeStruct((half, out_c), f32),
        )(g_rows, a_p, w2, b2f, wa1, ba1f, wa2)
        outs.append(out_h)

    out_t = jnp.concatenate(outs, axis=0)
    return out_t.T[None]
